# Initial kernel scaffold; baseline (speedup 1.0000x reference)
import jax
import jax.numpy as jnp
from jax.experimental import pallas as pl


def kernel(pix_to_face, bary_coords, faces, feature, shape_features, color, color_bg, aux_latent, W1, b1, W2, b2, W3, b3):
    raise NotImplementedError("write your pallas kernel here")


# speedup vs baseline: 8.4112x; 8.4112x over previous
import functools

import jax
import jax.numpy as jnp
from jax import lax
from jax.experimental import pallas as pl
from jax.experimental.pallas import tpu as pltpu
from jax.experimental.pallas import tpu_sc as plsc

B, H, W_IMG = 1, 256, 256
P = B * H * W_IMG
FD, SD = 256, 64
XD = FD + SD + 16
TBD = 384
HID = 512

NC, NS = 2, 16
NW = NC * NS
CHUNK = P // NW
BSZ = 64
NB = CHUNK // BSZ

BM = 2048


def _sc_gather_blend(p2f, f0, f1, f2, table, wv):
  mesh = plsc.VectorSubcoreMesh(core_axis_name="c", subcore_axis_name="s")

  @functools.partial(
      pl.kernel,
      out_type=jax.ShapeDtypeStruct((P, XD), jnp.float32),
      mesh=mesh,
      scratch_types=[
          pltpu.VMEM((NB, BSZ), jnp.int32),
          pltpu.VMEM((NB, BSZ), jnp.int32),
          pltpu.VMEM((NB, BSZ), jnp.int32),
          pltpu.VMEM((NB, BSZ), jnp.int32),
          pltpu.VMEM((CHUNK + 16,), jnp.float32),
          pltpu.VMEM((CHUNK + 16,), jnp.float32),
          pltpu.VMEM((CHUNK + 16,), jnp.float32),
          pltpu.VMEM((BSZ, TBD), jnp.float32),
          pltpu.VMEM((BSZ, TBD), jnp.float32),
          pltpu.VMEM((BSZ, TBD), jnp.float32),
          pltpu.VMEM((BSZ, XD), jnp.float32),
          pltpu.SemaphoreType.DMA,
      ],
  )
  def k(p2f_h, f0_h, f1_h, f2_h, tab_h, wv_h, x_h,
        pix, v0, v1, v2, wv0, wv1, wv2, g0, g1, g2, outb, sem):
    wid = lax.axis_index("s") * NC + lax.axis_index("c")
    base = wid * CHUNK
    pltpu.sync_copy(p2f_h.at[wid], pix)
    pltpu.sync_copy(wv_h.at[0, wid], wv0.at[pl.ds(0, CHUNK)])
    pltpu.sync_copy(wv_h.at[1, wid], wv1.at[pl.ds(0, CHUNK)])
    pltpu.sync_copy(wv_h.at[2, wid], wv2.at[pl.ds(0, CHUNK)])

    def face_fire(b, _):
      pltpu.async_copy(f0_h.at[pix.at[b]], v0.at[b], sem)
      pltpu.async_copy(f1_h.at[pix.at[b]], v1.at[b], sem)
      pltpu.async_copy(f2_h.at[pix.at[b]], v2.at[b], sem)
      return 0
    lax.fori_loop(0, NB, face_fire, 0)
    def face_drain(b, _):
      pltpu.make_async_copy(f0_h.at[pix.at[0]], v0.at[0], sem).wait()
      pltpu.make_async_copy(f1_h.at[pix.at[0]], v1.at[0], sem).wait()
      pltpu.make_async_copy(f2_h.at[pix.at[0]], v2.at[0], sem).wait()
      return 0
    lax.fori_loop(0, NB, face_drain, 0)

    def block(b, _):
      cps = [
          pltpu.async_copy(tab_h.at[v0.at[b]], g0, sem),
          pltpu.async_copy(tab_h.at[v1.at[b]], g1, sem),
          pltpu.async_copy(tab_h.at[v2.at[b]], g2, sem),
      ]
      for cp in cps:
        cp.wait()

      def row_fn(r, _):
        off = b * BSZ + r
        c0 = jnp.full((16,), wv0[pl.ds(off, 16)][0] * (1.0 / 3.0), jnp.float32)
        c1 = jnp.full((16,), wv1[pl.ds(off, 16)][0] * (1.0 / 3.0), jnp.float32)
        c2 = jnp.full((16,), wv2[pl.ds(off, 16)][0] * (1.0 / 3.0), jnp.float32)
        for kk in range(XD // 16):
          sl = pl.ds(kk * 16, 16)
          outb[r, sl] = g0[r, sl] * c0 + g1[r, sl] * c1 + g2[r, sl] * c2
        return 0

      lax.fori_loop(0, BSZ, row_fn, 0)
      pltpu.sync_copy(outb, x_h.at[pl.ds(base + b * BSZ, BSZ)])
      return 0

    lax.fori_loop(0, NB, block, 0)

  return k(p2f, f0, f1, f2, table, wv)


def _tc_body(p2f_ref, x_ref, w1_ref, b1_ref, w2_ref, b2_ref, w3_ref, b3_ref,
             aux_ref, bg_ref, o_ref):
  xin = x_ref[:, : FD + SD]
  cfm = x_ref[:, FD + SD : FD + SD + 3]
  w1a = w1_ref[: FD + SD, :]
  w1b = w1_ref[FD + SD :, :]
  auxh = jnp.dot(aux_ref[...], w1b, preferred_element_type=jnp.float32)
  h = jnp.dot(xin, w1a, preferred_element_type=jnp.float32)
  h = jax.nn.relu(h + b1_ref[...] + auxh)
  h = jnp.dot(h, w2_ref[...], preferred_element_type=jnp.float32)
  h = jax.nn.relu(h + b2_ref[...])
  o = jnp.dot(h, w3_ref[...], preferred_element_type=jnp.float32)
  o = jax.nn.sigmoid(o + b3_ref[...])
  colors = jnp.clip(o - 1.0 + cfm, -1.0, 1.0)
  mask = p2f_ref[...] > 0
  buf = jnp.where(mask, colors, bg_ref[...])
  o_ref[...] = jnp.concatenate([buf, mask.astype(jnp.float32)], axis=1)


def _tc_mlp(p2f2, x, W1, b1, W2, b2, W3, b3, aux, bg):
  full = lambda s: pl.BlockSpec(s, lambda i: (0, 0))
  return pl.pallas_call(
      _tc_body,
      grid=(P // BM,),
      in_specs=[
          pl.BlockSpec((BM, 1), lambda i: (i, 0)),
          pl.BlockSpec((BM, XD), lambda i: (i, 0)),
          full(W1.shape), full(b1.shape), full(W2.shape), full(b2.shape),
          full(W3.shape), full(b3.shape), full(aux.shape), full(bg.shape),
      ],
      out_specs=pl.BlockSpec((BM, 4), lambda i: (i, 0)),
      out_shape=jax.ShapeDtypeStruct((P, 4), jnp.float32),
  )(p2f2, x, W1, b1, W2, b2, W3, b3, aux, bg)


def kernel(pix_to_face, bary_coords, faces, feature, shape_features, color,
           color_bg, aux_latent, W1, b1, W2, b2, W3, b3):
  p2f = pix_to_face.reshape(NW, NB, BSZ)
  wv = bary_coords.reshape(P, 3).T.reshape(3, NW, CHUNK)
  f0 = faces[:, 0]
  f1 = faces[:, 1]
  f2 = faces[:, 2]
  V = feature.shape[0]
  table = jnp.concatenate(
      [feature, shape_features, color,
       jnp.zeros((V, TBD - FD - SD - 3), jnp.float32)], axis=1)
  x = _sc_gather_blend(p2f, f0, f1, f2, table, wv)
  out = _tc_mlp(pix_to_face.reshape(P, 1), x, W1, b1.reshape(1, HID), W2,
                b2.reshape(1, HID), W3, b3.reshape(1, 3), aux_latent,
                color_bg[:, :, 0, 0])
  return out.reshape(B, H, W_IMG, 4)


# speedup vs baseline: 9.9198x; 1.1794x over previous
import functools

import jax
import jax.numpy as jnp
from jax import lax
from jax.experimental import pallas as pl
from jax.experimental.pallas import tpu as pltpu
from jax.experimental.pallas import tpu_sc as plsc

B, H, W_IMG = 1, 256, 256
P = B * H * W_IMG
FD, SD = 256, 64
XD = FD + SD + 16
TBD = 384
HID = 512

NC, NS = 2, 16
NW = NC * NS
CHUNK = P // NW
BSZ = 32
NB = CHUNK // BSZ

BM = 2048


def _sc_gather_blend(p2f, f0, f1, f2, table, wv):
  mesh = plsc.VectorSubcoreMesh(core_axis_name="c", subcore_axis_name="s")

  @functools.partial(
      pl.kernel,
      out_type=jax.ShapeDtypeStruct((P, XD), jnp.float32),
      mesh=mesh,
      scratch_types=[
          pltpu.VMEM((NB, BSZ), jnp.int32),
          pltpu.VMEM((NB, BSZ), jnp.int32),
          pltpu.VMEM((NB, BSZ), jnp.int32),
          pltpu.VMEM((NB, BSZ), jnp.int32),
          pltpu.VMEM((CHUNK + 16,), jnp.float32),
          pltpu.VMEM((CHUNK + 16,), jnp.float32),
          pltpu.VMEM((CHUNK + 16,), jnp.float32),
          pltpu.VMEM((BSZ, TBD), jnp.float32),
          pltpu.VMEM((BSZ, TBD), jnp.float32),
          pltpu.VMEM((BSZ, TBD), jnp.float32),
          pltpu.VMEM((BSZ, TBD), jnp.float32),
          pltpu.VMEM((BSZ, TBD), jnp.float32),
          pltpu.VMEM((BSZ, TBD), jnp.float32),
          pltpu.VMEM((BSZ, XD), jnp.float32),
          pltpu.SemaphoreType.DMA,
          pltpu.SemaphoreType.DMA,
          pltpu.SemaphoreType.DMA,
      ],
  )
  def k(p2f_h, f0_h, f1_h, f2_h, tab_h, wv_h, x_h,
        pix, v0, v1, v2, wv0, wv1, wv2,
        ga0, ga1, ga2, gb0, gb1, gb2, outb, semA, semB, semF):
    wid = lax.axis_index("s") * NC + lax.axis_index("c")
    base = wid * CHUNK
    pltpu.sync_copy(p2f_h.at[wid], pix)
    pltpu.sync_copy(wv_h.at[0, wid], wv0.at[pl.ds(0, CHUNK)])
    pltpu.sync_copy(wv_h.at[1, wid], wv1.at[pl.ds(0, CHUNK)])
    pltpu.sync_copy(wv_h.at[2, wid], wv2.at[pl.ds(0, CHUNK)])

    def face_fire(b, _):
      pltpu.async_copy(f0_h.at[pix.at[b]], v0.at[b], semF)
      pltpu.async_copy(f1_h.at[pix.at[b]], v1.at[b], semF)
      pltpu.async_copy(f2_h.at[pix.at[b]], v2.at[b], semF)
      return 0
    lax.fori_loop(0, NB, face_fire, 0)
    def face_drain(b, _):
      pltpu.make_async_copy(f0_h.at[pix.at[0]], v0.at[0], semF).wait()
      pltpu.make_async_copy(f1_h.at[pix.at[0]], v1.at[0], semF).wait()
      pltpu.make_async_copy(f2_h.at[pix.at[0]], v2.at[0], semF).wait()
      return 0
    lax.fori_loop(0, NB, face_drain, 0)

    def fire(b, bufs, sem):
      g0, g1, g2 = bufs
      pltpu.async_copy(tab_h.at[v0.at[b]], g0, sem)
      pltpu.async_copy(tab_h.at[v1.at[b]], g1, sem)
      pltpu.async_copy(tab_h.at[v2.at[b]], g2, sem)

    def drain(bufs, sem):
      g0, g1, g2 = bufs
      pltpu.make_async_copy(tab_h.at[v0.at[0]], g0, sem).wait()
      pltpu.make_async_copy(tab_h.at[v1.at[0]], g1, sem).wait()
      pltpu.make_async_copy(tab_h.at[v2.at[0]], g2, sem).wait()

    def blend(b, bufs):
      g0, g1, g2 = bufs
      def row_fn(r, _):
        off = b * BSZ + r
        c0 = jnp.full((16,), wv0[pl.ds(off, 16)][0], jnp.float32)
        c1 = jnp.full((16,), wv1[pl.ds(off, 16)][0], jnp.float32)
        c2 = jnp.full((16,), wv2[pl.ds(off, 16)][0], jnp.float32)
        for kk in range(XD // 16):
          sl = pl.ds(kk * 16, 16)
          outb[r, sl] = g0[r, sl] * c0 + g1[r, sl] * c1 + g2[r, sl] * c2
        return 0
      lax.fori_loop(0, BSZ, row_fn, 0)
      pltpu.sync_copy(outb, x_h.at[pl.ds(base + b * BSZ, BSZ)])

    ring_a = (ga0, ga1, ga2)
    ring_b = (gb0, gb1, gb2)

    fire(0, ring_a, semA)
    def pair(g, _):
      ba = 2 * g
      fire(ba + 1, ring_b, semB)
      drain(ring_a, semA)
      blend(ba, ring_a)
      @pl.when(ba + 2 < NB)
      def _():
        fire(ba + 2, ring_a, semA)
      drain(ring_b, semB)
      blend(ba + 1, ring_b)
      return 0
    lax.fori_loop(0, NB // 2, pair, 0)

  return k(p2f, f0, f1, f2, table, wv)


def _tc_body(p2f_ref, x_ref, w1_ref, b1_ref, w2_ref, b2_ref, w3_ref, b3_ref,
             aux_ref, bg_ref, o_ref):
  xin = x_ref[:, : FD + SD].astype(jnp.bfloat16)
  cfm = x_ref[:, FD + SD : FD + SD + 3]
  w1a = w1_ref[: FD + SD, :]
  w1b = w1_ref[FD + SD :, :]
  auxh = jnp.dot(aux_ref[...], w1b, preferred_element_type=jnp.float32)
  h = jnp.dot(xin, w1a, preferred_element_type=jnp.float32)
  h = jax.nn.relu(h + b1_ref[...] + auxh)
  h = jnp.dot(h.astype(jnp.bfloat16), w2_ref[...],
              preferred_element_type=jnp.float32)
  h = jax.nn.relu(h + b2_ref[...])
  o = jnp.dot(h.astype(jnp.bfloat16), w3_ref[...],
              preferred_element_type=jnp.float32)
  o = jax.nn.sigmoid(o + b3_ref[...])
  colors = jnp.clip(o - 1.0 + cfm, -1.0, 1.0)
  mask = p2f_ref[...] > 0
  buf = jnp.where(mask, colors, bg_ref[...])
  o_ref[...] = jnp.concatenate([buf, mask.astype(jnp.float32)], axis=1)


def _tc_mlp(p2f2, x, W1, b1, W2, b2, W3, b3, aux, bg):
  full = lambda s: pl.BlockSpec(s, lambda i: (0, 0))
  return pl.pallas_call(
      _tc_body,
      grid=(P // BM,),
      in_specs=[
          pl.BlockSpec((BM, 1), lambda i: (i, 0)),
          pl.BlockSpec((BM, XD), lambda i: (i, 0)),
          full(W1.shape), full(b1.shape), full(W2.shape), full(b2.shape),
          full(W3.shape), full(b3.shape), full(aux.shape), full(bg.shape),
      ],
      out_specs=pl.BlockSpec((BM, 4), lambda i: (i, 0)),
      out_shape=jax.ShapeDtypeStruct((P, 4), jnp.float32),
  )(p2f2, x, W1, b1, W2, b2, W3, b3, aux, bg)


def kernel(pix_to_face, bary_coords, faces, feature, shape_features, color,
           color_bg, aux_latent, W1, b1, W2, b2, W3, b3):
  bf = jnp.bfloat16
  p2f = pix_to_face.reshape(NW, NB, BSZ)
  wv = (bary_coords.reshape(P, 3).T * (1.0 / 3.0)).reshape(3, NW, CHUNK)
  f0 = faces[:, 0]
  f1 = faces[:, 1]
  f2 = faces[:, 2]
  V = feature.shape[0]
  table = jnp.concatenate(
      [feature, shape_features, color,
       jnp.zeros((V, TBD - FD - SD - 3), jnp.float32)], axis=1)
  x = _sc_gather_blend(p2f, f0, f1, f2, table, wv)
  out = _tc_mlp(pix_to_face.reshape(P, 1), x, W1.astype(bf),
                b1.reshape(1, HID), W2.astype(bf), b2.reshape(1, HID),
                W3.astype(bf), b3.reshape(1, 3), aux_latent.astype(bf),
                color_bg[:, :, 0, 0])
  return out.reshape(B, H, W_IMG, 4)
